# baseline (speedup 1.0000x reference)
import functools

import jax
import jax.numpy as jnp
from jax import lax
from jax.experimental import pallas as pl
from jax.experimental.pallas import tpu as pltpu
from jax.experimental.pallas import tpu_sc as plsc

EMB_DIM = 128
NUM_KEYS = 100000
BATCH = 4096
TOP_M = 32
TEMP = 0.07

NPAD = 102400
G = 32
NG = NPAD // G
BQA = 256
BKA = 4096
BQS = 256
B2 = 2 * BATCH
NEG = -1e30


def _qn_kernel(q_ref, wp_ref, bp_ref, out_ref):
    pq = lax.dot_general(q_ref[...], wp_ref[0],
                         (((1,), (1,)), ((), ())),
                         preferred_element_type=jnp.float32)
    pq = pq + bp_ref[0]
    nrm = jnp.sqrt(jnp.sum(pq * pq, axis=1, keepdims=True))
    out_ref[0] = pq / jnp.maximum(nrm, 1e-12)


def _keysn_kernel(b_ref, out_ref):
    b = b_ref[...]
    nrm = jnp.sqrt(jnp.sum(b * b, axis=1, keepdims=True))
    out_ref[...] = b / jnp.maximum(nrm, 1e-12)


def _score_kernel(qn_ref, bank_ref, s_ref, gm_ref):
    j = pl.program_id(0)
    s = lax.dot_general(qn_ref[...], bank_ref[...],
                        (((1,), (1,)), ((), ())),
                        preferred_element_type=jnp.float32)
    kidx = j * BKA + lax.broadcasted_iota(jnp.int32, (1, BKA), 1)
    s = jnp.where(kidx < NUM_KEYS, s, NEG)
    s_ref[...] = s
    gm_ref[...] = jnp.max(s.reshape(BQA, BKA // G, G), axis=2)


def _sel1_kernel(gm_ref, sidx_ref, idx1_ref):
    i = pl.program_id(0)
    io = lax.broadcasted_iota(jnp.int32, (BQS, NG), 1)
    iom = lax.broadcasted_iota(jnp.int32, (BQS, TOP_M), 1)

    def body(t, carry):
        gm, acc = carry
        a = jnp.argmax(gm, axis=1).astype(jnp.int32)
        acc = jnp.where(iom == t, a[:, None], acc)
        gm = jnp.where(io == a[:, None], NEG, gm)
        return gm, acc

    _, idx1 = lax.fori_loop(
        0, TOP_M, body,
        (gm_ref[...], jnp.zeros((BQS, TOP_M), jnp.int32)))
    rows = i * BQS + lax.broadcasted_iota(jnp.int32, (BQS, TOP_M), 0)
    idx1_ref[...] = idx1
    sidx_ref[...] = rows * (NPAD // EMB_DIM) + idx1 // 4


def _sel2_kernel(ch_ref, idx1_ref, ts_ref, ti_ref):
    idx1 = idx1_ref[...]
    sub = (idx1 % 4)[:, :, None] * G + lax.broadcasted_iota(
        jnp.int32, (BQS, TOP_M, G), 2)
    c = jnp.take_along_axis(ch_ref[...], sub, axis=2).reshape(BQS, TOP_M * G)
    ioc = lax.broadcasted_iota(jnp.int32, (BQS, TOP_M * G), 1)
    iom = lax.broadcasted_iota(jnp.int32, (BQS, TOP_M), 1)

    def body(t, carry):
        c, ts, ti = carry
        v = jnp.max(c, axis=1)
        p = jnp.argmax(c, axis=1).astype(jnp.int32)
        grp = jnp.take_along_axis(idx1, (p // G)[:, None], axis=1)[:, 0]
        eid = grp * G + p % G
        ts = jnp.where(iom == t, v[:, None], ts)
        ti = jnp.where(iom == t, eid[:, None], ti)
        c = jnp.where(ioc == p[:, None], NEG, c)
        return c, ts, ti

    _, ts, ti = lax.fori_loop(
        0, TOP_M, body,
        (c, jnp.zeros((BQS, TOP_M), jnp.float32),
         jnp.zeros((BQS, TOP_M), jnp.int32)))
    ts_ref[...] = ts
    ti_ref[...] = ti


def _fin_kernel(s0_ref, s1_ref, v0_ref, v1_ref, wu0_ref, bu0_ref,
                wu1_ref, bu1_ref, g_ref, b_ref, o_ref):
    def zsum(s, v):
        x = s / TEMP
        m = jnp.max(x, axis=1, keepdims=True)
        e = jnp.exp(x - m)
        a = e / jnp.sum(e, axis=1, keepdims=True)
        return jnp.sum(v * a[:, :, None], axis=1)

    z0 = zsum(s0_ref[...], v0_ref[...])
    z1 = zsum(s1_ref[...], v1_ref[...])
    r = (lax.dot_general(z0, wu0_ref[...], (((1,), (1,)), ((), ())),
                         preferred_element_type=jnp.float32)
         + bu0_ref[0][None, :]
         + lax.dot_general(z1, wu1_ref[...], (((1,), (1,)), ((), ())),
                           preferred_element_type=jnp.float32)
         + bu1_ref[0][None, :])
    mu = jnp.mean(r, axis=1, keepdims=True)
    var = jnp.mean((r - mu) * (r - mu), axis=1, keepdims=True)
    o_ref[...] = ((r - mu) / jnp.sqrt(var + 1e-5)
                  * g_ref[0][None, :] + b_ref[0][None, :])


def _sc_gather(table, idx, d, chunk):
    n = idx.shape[0]
    info = plsc.get_sparse_core_info()
    nw = info.num_cores * info.num_subcores
    b_per_w = n // nw
    mesh = plsc.VectorSubcoreMesh(core_axis_name="c", subcore_axis_name="s")

    @functools.partial(
        pl.kernel, mesh=mesh,
        out_type=jax.ShapeDtypeStruct((n, d), jnp.float32),
        scratch_types=[
            pltpu.VMEM((chunk,), jnp.int32),
            pltpu.VMEM((chunk, d), jnp.float32),
            pltpu.SemaphoreType.DMA,
        ],
    )
    def k(table_hbm, idx_hbm, out_hbm, idx_v, rows_v, sem):
        wid = lax.axis_index("s") * info.num_cores + lax.axis_index("c")
        base = wid * b_per_w
        for c in range(b_per_w // chunk):
            off = base + c * chunk
            pltpu.sync_copy(idx_hbm.at[pl.ds(off, chunk)], idx_v)
            pltpu.async_copy(table_hbm.at[idx_v], rows_v, sem).wait()
            pltpu.sync_copy(rows_v, out_hbm.at[pl.ds(off, chunk)])

    return k(table, idx)


def kernel(q, bank_embs, Wp0, bp0, Wp1, bp1, Wu0, bu0, Wu1, bu1,
           gamma, beta, top_m):
    f32 = jnp.float32
    bank_pad = jnp.pad(bank_embs, ((0, NPAD - NUM_KEYS), (0, 0)))
    wp = jnp.stack([Wp0, Wp1])
    bp = jnp.stack([bp0, bp1]).reshape(2, 1, EMB_DIM)

    qn = pl.pallas_call(
        _qn_kernel,
        grid=(2,),
        in_specs=[
            pl.BlockSpec((BATCH, EMB_DIM), lambda s: (0, 0)),
            pl.BlockSpec((1, EMB_DIM, EMB_DIM), lambda s: (s, 0, 0)),
            pl.BlockSpec((1, 1, EMB_DIM), lambda s: (s, 0, 0)),
        ],
        out_specs=pl.BlockSpec((1, BATCH, EMB_DIM), lambda s: (s, 0, 0)),
        out_shape=jax.ShapeDtypeStruct((2, BATCH, EMB_DIM), f32),
    )(q, wp, bp).reshape(B2, EMB_DIM)

    keys_n = pl.pallas_call(
        _keysn_kernel,
        grid=(NPAD // BKA,),
        in_specs=[pl.BlockSpec((BKA, EMB_DIM), lambda j: (j, 0))],
        out_specs=pl.BlockSpec((BKA, EMB_DIM), lambda j: (j, 0)),
        out_shape=jax.ShapeDtypeStruct((NPAD, EMB_DIM), f32),
    )(bank_pad)

    scores, gm1 = pl.pallas_call(
        _score_kernel,
        grid=(NPAD // BKA, B2 // BQA),
        in_specs=[
            pl.BlockSpec((BQA, EMB_DIM), lambda j, i: (i, 0)),
            pl.BlockSpec((BKA, EMB_DIM), lambda j, i: (j, 0)),
        ],
        out_specs=[
            pl.BlockSpec((BQA, BKA), lambda j, i: (i, j)),
            pl.BlockSpec((BQA, BKA // G), lambda j, i: (i, j)),
        ],
        out_shape=[
            jax.ShapeDtypeStruct((B2, NPAD), f32),
            jax.ShapeDtypeStruct((B2, NG), f32),
        ],
    )(qn, keys_n)

    sidx, idx1 = pl.pallas_call(
        _sel1_kernel,
        grid=(B2 // BQS,),
        in_specs=[pl.BlockSpec((BQS, NG), lambda i: (i, 0))],
        out_specs=[
            pl.BlockSpec((BQS, TOP_M), lambda i: (i, 0)),
            pl.BlockSpec((BQS, TOP_M), lambda i: (i, 0)),
        ],
        out_shape=[
            jax.ShapeDtypeStruct((B2, TOP_M), jnp.int32),
            jax.ShapeDtypeStruct((B2, TOP_M), jnp.int32),
        ],
    )(gm1)

    chunks = _sc_gather(scores.reshape(B2 * NPAD // EMB_DIM, EMB_DIM),
                        sidx.reshape(B2 * TOP_M), EMB_DIM, 512)

    top_s, top_i = pl.pallas_call(
        _sel2_kernel,
        grid=(B2 // BQS,),
        in_specs=[
            pl.BlockSpec((BQS, TOP_M, EMB_DIM), lambda i: (i, 0, 0)),
            pl.BlockSpec((BQS, TOP_M), lambda i: (i, 0)),
        ],
        out_specs=[
            pl.BlockSpec((BQS, TOP_M), lambda i: (i, 0)),
            pl.BlockSpec((BQS, TOP_M), lambda i: (i, 0)),
        ],
        out_shape=[
            jax.ShapeDtypeStruct((B2, TOP_M), f32),
            jax.ShapeDtypeStruct((B2, TOP_M), jnp.int32),
        ],
    )(chunks.reshape(B2, TOP_M, EMB_DIM), idx1)

    vals = _sc_gather(bank_embs, top_i.reshape(B2 * TOP_M), EMB_DIM, 512)
    vals = vals.reshape(2, BATCH, TOP_M, EMB_DIM)

    out = pl.pallas_call(
        _fin_kernel,
        grid=(BATCH // BQS,),
        in_specs=[
            pl.BlockSpec((BQS, TOP_M), lambda i: (i, 0)),
            pl.BlockSpec((BQS, TOP_M), lambda i: (i, 0)),
            pl.BlockSpec((BQS, TOP_M, EMB_DIM), lambda i: (i, 0, 0)),
            pl.BlockSpec((BQS, TOP_M, EMB_DIM), lambda i: (i, 0, 0)),
            pl.BlockSpec((EMB_DIM, EMB_DIM), lambda i: (0, 0)),
            pl.BlockSpec((1, EMB_DIM), lambda i: (0, 0)),
            pl.BlockSpec((EMB_DIM, EMB_DIM), lambda i: (0, 0)),
            pl.BlockSpec((1, EMB_DIM), lambda i: (0, 0)),
            pl.BlockSpec((1, EMB_DIM), lambda i: (0, 0)),
            pl.BlockSpec((1, EMB_DIM), lambda i: (0, 0)),
        ],
        out_specs=pl.BlockSpec((BQS, EMB_DIM), lambda i: (i, 0)),
        out_shape=jax.ShapeDtypeStruct((BATCH, EMB_DIM), f32),
    )(top_s[:BATCH], top_s[BATCH:], vals[0], vals[1],
      Wu0, bu0.reshape(1, EMB_DIM), Wu1, bu1.reshape(1, EMB_DIM),
      gamma.reshape(1, EMB_DIM), beta.reshape(1, EMB_DIM))
    return out


# speedup vs baseline: 1.1939x; 1.1939x over previous
import functools

import jax
import jax.numpy as jnp
from jax import lax
from jax.experimental import pallas as pl
from jax.experimental.pallas import tpu as pltpu
from jax.experimental.pallas import tpu_sc as plsc

EMB_DIM = 128
NUM_KEYS = 100000
BATCH = 4096
TOP_M = 32
TEMP = 0.07

NPAD = 102400
G = 32
NG = NPAD // G
BQA = 256
BKA = 4096
BQS = 256
B2 = 2 * BATCH
NEG = -1e30


def _qn_kernel(q_ref, wp_ref, bp_ref, out_ref):
    pq = lax.dot_general(q_ref[...], wp_ref[0],
                         (((1,), (1,)), ((), ())),
                         preferred_element_type=jnp.float32)
    pq = pq + bp_ref[0]
    nrm = jnp.sqrt(jnp.sum(pq * pq, axis=1, keepdims=True))
    out_ref[0] = pq / jnp.maximum(nrm, 1e-12)


def _keysn_kernel(b_ref, out_ref):
    b = b_ref[...]
    nrm = jnp.sqrt(jnp.sum(b * b, axis=1, keepdims=True))
    out_ref[...] = b / jnp.maximum(nrm, 1e-12)


def _score_kernel(qn_ref, bank_ref, s_ref, gm_ref):
    j = pl.program_id(0)
    s = lax.dot_general(qn_ref[...], bank_ref[...],
                        (((1,), (1,)), ((), ())),
                        preferred_element_type=jnp.float32)
    kidx = j * BKA + lax.broadcasted_iota(jnp.int32, (1, BKA), 1)
    s = jnp.where(kidx < NUM_KEYS, s, NEG)
    s_ref[...] = s.reshape(BQA, BKA // EMB_DIM, EMB_DIM)
    gm_ref[...] = jnp.max(s.reshape(BQA, BKA // G, G), axis=2)


def _sel1_kernel(gm_ref, sidx_ref, idx1_ref):
    i = pl.program_id(0)
    io = lax.broadcasted_iota(jnp.int32, (BQS, NG), 1)
    iom = lax.broadcasted_iota(jnp.int32, (BQS, TOP_M), 1)

    def body(t, carry):
        gm, acc = carry
        a = jnp.argmax(gm, axis=1).astype(jnp.int32)
        acc = jnp.where(iom == t, a[:, None], acc)
        gm = jnp.where(io == a[:, None], NEG, gm)
        return gm, acc

    _, idx1 = lax.fori_loop(
        0, TOP_M, body,
        (gm_ref[...], jnp.zeros((BQS, TOP_M), jnp.int32)))
    rows = i * BQS + lax.broadcasted_iota(jnp.int32, (BQS, TOP_M), 0)
    idx1_ref[...] = idx1
    sidx_ref[...] = rows * (NPAD // EMB_DIM) + idx1 // 4


def _sel2_kernel(ch_ref, idx1_ref, ts_ref, ti_ref):
    idx1 = idx1_ref[...]
    sub = (idx1 % 4)[:, :, None] * G + lax.broadcasted_iota(
        jnp.int32, (BQS, TOP_M, G), 2)
    c = jnp.take_along_axis(ch_ref[...], sub, axis=2).reshape(BQS, TOP_M * G)
    ioc = lax.broadcasted_iota(jnp.int32, (BQS, TOP_M * G), 1)
    iom = lax.broadcasted_iota(jnp.int32, (BQS, TOP_M), 1)

    def body(t, carry):
        c, ts, ti = carry
        v = jnp.max(c, axis=1)
        p = jnp.argmax(c, axis=1).astype(jnp.int32)
        grp = jnp.take_along_axis(idx1, (p // G)[:, None], axis=1)[:, 0]
        eid = grp * G + p % G
        ts = jnp.where(iom == t, v[:, None], ts)
        ti = jnp.where(iom == t, eid[:, None], ti)
        c = jnp.where(ioc == p[:, None], NEG, c)
        return c, ts, ti

    _, ts, ti = lax.fori_loop(
        0, TOP_M, body,
        (c, jnp.zeros((BQS, TOP_M), jnp.float32),
         jnp.zeros((BQS, TOP_M), jnp.int32)))
    ts_ref[...] = ts
    ti_ref[...] = ti


def _fin_kernel(s0_ref, s1_ref, v0_ref, v1_ref, wu0_ref, bu0_ref,
                wu1_ref, bu1_ref, g_ref, b_ref, o_ref):
    def zsum(s, v):
        x = s / TEMP
        m = jnp.max(x, axis=1, keepdims=True)
        e = jnp.exp(x - m)
        a = e / jnp.sum(e, axis=1, keepdims=True)
        return jnp.sum(v * a[:, :, None], axis=1)

    z0 = zsum(s0_ref[...], v0_ref[...])
    z1 = zsum(s1_ref[...], v1_ref[...])
    r = (lax.dot_general(z0, wu0_ref[...], (((1,), (1,)), ((), ())),
                         preferred_element_type=jnp.float32)
         + bu0_ref[0][None, :]
         + lax.dot_general(z1, wu1_ref[...], (((1,), (1,)), ((), ())),
                           preferred_element_type=jnp.float32)
         + bu1_ref[0][None, :])
    mu = jnp.mean(r, axis=1, keepdims=True)
    var = jnp.mean((r - mu) * (r - mu), axis=1, keepdims=True)
    o_ref[...] = ((r - mu) / jnp.sqrt(var + 1e-5)
                  * g_ref[0][None, :] + b_ref[0][None, :])


def _sc_gather(table, idx, d, chunk):
    n = idx.shape[0]
    info = plsc.get_sparse_core_info()
    nw = info.num_cores * info.num_subcores
    b_per_w = n // nw
    mesh = plsc.VectorSubcoreMesh(core_axis_name="c", subcore_axis_name="s")

    @functools.partial(
        pl.kernel, mesh=mesh,
        out_type=jax.ShapeDtypeStruct((n, d), jnp.float32),
        scratch_types=[
            pltpu.VMEM((chunk,), jnp.int32),
            pltpu.VMEM((chunk, d), jnp.float32),
            pltpu.SemaphoreType.DMA,
        ],
    )
    def k(table_hbm, idx_hbm, out_hbm, idx_v, rows_v, sem):
        wid = lax.axis_index("s") * info.num_cores + lax.axis_index("c")
        base = wid * b_per_w
        for c in range(b_per_w // chunk):
            off = base + c * chunk
            pltpu.sync_copy(idx_hbm.at[pl.ds(off, chunk)], idx_v)
            pltpu.async_copy(table_hbm.at[idx_v], rows_v, sem).wait()
            pltpu.sync_copy(rows_v, out_hbm.at[pl.ds(off, chunk)])

    return k(table, idx)


def kernel(q, bank_embs, Wp0, bp0, Wp1, bp1, Wu0, bu0, Wu1, bu1,
           gamma, beta, top_m):
    f32 = jnp.float32
    bank_pad = jnp.pad(bank_embs, ((0, NPAD - NUM_KEYS), (0, 0)))
    wp = jnp.stack([Wp0, Wp1])
    bp = jnp.stack([bp0, bp1]).reshape(2, 1, EMB_DIM)

    qn = pl.pallas_call(
        _qn_kernel,
        grid=(2,),
        in_specs=[
            pl.BlockSpec((BATCH, EMB_DIM), lambda s: (0, 0)),
            pl.BlockSpec((1, EMB_DIM, EMB_DIM), lambda s: (s, 0, 0)),
            pl.BlockSpec((1, 1, EMB_DIM), lambda s: (s, 0, 0)),
        ],
        out_specs=pl.BlockSpec((1, BATCH, EMB_DIM), lambda s: (s, 0, 0)),
        out_shape=jax.ShapeDtypeStruct((2, BATCH, EMB_DIM), f32),
    )(q, wp, bp).reshape(B2, EMB_DIM)

    keys_n = pl.pallas_call(
        _keysn_kernel,
        grid=(NPAD // BKA,),
        in_specs=[pl.BlockSpec((BKA, EMB_DIM), lambda j: (j, 0))],
        out_specs=pl.BlockSpec((BKA, EMB_DIM), lambda j: (j, 0)),
        out_shape=jax.ShapeDtypeStruct((NPAD, EMB_DIM), f32),
    )(bank_pad)

    scores, gm1 = pl.pallas_call(
        _score_kernel,
        grid=(NPAD // BKA, B2 // BQA),
        in_specs=[
            pl.BlockSpec((BQA, EMB_DIM), lambda j, i: (i, 0)),
            pl.BlockSpec((BKA, EMB_DIM), lambda j, i: (j, 0)),
        ],
        out_specs=[
            pl.BlockSpec((BQA, BKA // EMB_DIM, EMB_DIM),
                         lambda j, i: (i, j, 0)),
            pl.BlockSpec((BQA, BKA // G), lambda j, i: (i, j)),
        ],
        out_shape=[
            jax.ShapeDtypeStruct((B2, NPAD // EMB_DIM, EMB_DIM), f32),
            jax.ShapeDtypeStruct((B2, NG), f32),
        ],
    )(qn, keys_n)

    sidx, idx1 = pl.pallas_call(
        _sel1_kernel,
        grid=(B2 // BQS,),
        in_specs=[pl.BlockSpec((BQS, NG), lambda i: (i, 0))],
        out_specs=[
            pl.BlockSpec((BQS, TOP_M), lambda i: (i, 0)),
            pl.BlockSpec((BQS, TOP_M), lambda i: (i, 0)),
        ],
        out_shape=[
            jax.ShapeDtypeStruct((B2, TOP_M), jnp.int32),
            jax.ShapeDtypeStruct((B2, TOP_M), jnp.int32),
        ],
    )(gm1)

    chunks = _sc_gather(scores.reshape(B2 * NPAD // EMB_DIM, EMB_DIM),
                        sidx.reshape(B2 * TOP_M), EMB_DIM, 512)

    top_s, top_i = pl.pallas_call(
        _sel2_kernel,
        grid=(B2 // BQS,),
        in_specs=[
            pl.BlockSpec((BQS, TOP_M, EMB_DIM), lambda i: (i, 0, 0)),
            pl.BlockSpec((BQS, TOP_M), lambda i: (i, 0)),
        ],
        out_specs=[
            pl.BlockSpec((BQS, TOP_M), lambda i: (i, 0)),
            pl.BlockSpec((BQS, TOP_M), lambda i: (i, 0)),
        ],
        out_shape=[
            jax.ShapeDtypeStruct((B2, TOP_M), f32),
            jax.ShapeDtypeStruct((B2, TOP_M), jnp.int32),
        ],
    )(chunks.reshape(B2, TOP_M, EMB_DIM), idx1)

    vals = _sc_gather(bank_embs, top_i.reshape(B2 * TOP_M), EMB_DIM, 512)
    vals = vals.reshape(2, BATCH, TOP_M, EMB_DIM)

    out = pl.pallas_call(
        _fin_kernel,
        grid=(BATCH // BQS,),
        in_specs=[
            pl.BlockSpec((BQS, TOP_M), lambda i: (i, 0)),
            pl.BlockSpec((BQS, TOP_M), lambda i: (i, 0)),
            pl.BlockSpec((BQS, TOP_M, EMB_DIM), lambda i: (i, 0, 0)),
            pl.BlockSpec((BQS, TOP_M, EMB_DIM), lambda i: (i, 0, 0)),
            pl.BlockSpec((EMB_DIM, EMB_DIM), lambda i: (0, 0)),
            pl.BlockSpec((1, EMB_DIM), lambda i: (0, 0)),
            pl.BlockSpec((EMB_DIM, EMB_DIM), lambda i: (0, 0)),
            pl.BlockSpec((1, EMB_DIM), lambda i: (0, 0)),
            pl.BlockSpec((1, EMB_DIM), lambda i: (0, 0)),
            pl.BlockSpec((1, EMB_DIM), lambda i: (0, 0)),
        ],
        out_specs=pl.BlockSpec((BQS, EMB_DIM), lambda i: (i, 0)),
        out_shape=jax.ShapeDtypeStruct((BATCH, EMB_DIM), f32),
    )(top_s[:BATCH], top_s[BATCH:], vals[0], vals[1],
      Wu0, bu0.reshape(1, EMB_DIM), Wu1, bu1.reshape(1, EMB_DIM),
      gamma.reshape(1, EMB_DIM), beta.reshape(1, EMB_DIM))
    return out
